# baseline (speedup 1.0000x reference)
import functools

import jax
import jax.numpy as jnp
from jax import lax
from jax.experimental import pallas as pl
from jax.experimental.pallas import tpu as pltpu
from jax.experimental.pallas import tpu_sc as plsc

N = 10000
E = 320000
D = 128

NC = 2
NS = 16
NW = NC * NS
EW = E // NW
CB = 80
FULL = EW // CB
TAIL = EW - FULL * CB
RPT = 624
RPT_LAST = N - RPT * (NS - 1)

NBUF = 3
STEPS = FULL // NBUF - 1


@functools.cache
def _make_sc_aggregate():
    mesh = plsc.VectorSubcoreMesh(
        core_axis_name="c", subcore_axis_name="s",
        num_cores=NC, num_subcores=NS,
    )

    @functools.partial(
        pl.kernel,
        out_type=jax.ShapeDtypeStruct((NC, N, D), jnp.float32),
        mesh=mesh,
        scratch_types=[
            pltpu.VMEM((FULL * CB,), jnp.int32),
            [pltpu.VMEM((CB,), jnp.int32) for _ in range(NBUF)],
            [pltpu.VMEM((CB, D), jnp.float32) for _ in range(NBUF)],
            [pltpu.SemaphoreType.DMA for _ in range(NBUF)],
            [pltpu.SemaphoreType.DMA for _ in range(NBUF)],
        ] + ([
            pltpu.VMEM((TAIL,), jnp.int32),
            pltpu.VMEM((TAIL,), jnp.int32),
            pltpu.VMEM((TAIL, D), jnp.float32),
            pltpu.SemaphoreType.DMA,
        ] if TAIL else []) + [
            pltpu.VMEM_SHARED((N, D), jnp.float32),
        ],
    )
    def _sc_aggregate(x_hbm, edge_hbm, zeros_hbm, out_hbm, src_v, dst_v,
                      rows, gsem, isem, *rest):
        if TAIL:
            src_t, dst_t, rows_t, sem, acc_sh = rest
        else:
            acc_sh, = rest
        cid = lax.axis_index("c")
        sid = lax.axis_index("s")
        wid = cid * NS + sid
        ebase = wid * EW

        pltpu.sync_copy(edge_hbm.at[pl.ds(ebase, FULL * CB)], src_v)
        if TAIL:
            pltpu.sync_copy(edge_hbm.at[pl.ds(ebase + FULL * CB, TAIL)],
                            src_t)
            pltpu.sync_copy(edge_hbm.at[pl.ds(E + ebase + FULL * CB, TAIL)],
                            dst_t)

        def issue(b, i):
            pltpu.async_copy(edge_hbm.at[pl.ds(E + ebase + i * CB, CB)],
                             dst_v[b], isem[b])
            pltpu.async_copy(x_hbm.at[src_v.at[pl.ds(i * CB, CB)]],
                             rows[b], gsem[b])

        def consume(b, i):
            pltpu.make_async_copy(edge_hbm.at[pl.ds(E + ebase + i * CB, CB)],
                                  dst_v[b], isem[b]).wait()
            pltpu.make_async_copy(x_hbm.at[src_v.at[pl.ds(i * CB, CB)]],
                                  rows[b], gsem[b]).wait()
            pltpu.sync_copy(rows[b], acc_sh.at[dst_v[b]], add=True)

        for b in range(NBUF):
            issue(b, b)
        if TAIL:
            pltpu.async_copy(x_hbm.at[src_t], rows_t, sem)

        @pl.when(sid < NS - 1)
        def _():
            pltpu.sync_copy(zeros_hbm.at[pl.ds(0, RPT)],
                            acc_sh.at[pl.ds(sid * RPT, RPT)])

        @pl.when(sid == NS - 1)
        def _():
            pltpu.sync_copy(zeros_hbm,
                            acc_sh.at[pl.ds((NS - 1) * RPT, RPT_LAST)])

        plsc.subcore_barrier()

        @pl.loop(0, STEPS)
        def _(j):
            i0 = j * NBUF
            for b in range(NBUF):
                consume(b, i0 + b)
                issue(b, i0 + NBUF + b)

        for k in range(STEPS * NBUF, FULL):
            b = k % NBUF
            consume(b, k)
            if k + NBUF < FULL:
                issue(b, k + NBUF)

        if TAIL:
            pltpu.make_async_copy(x_hbm.at[src_t], rows_t, sem).wait()
            pltpu.sync_copy(rows_t, acc_sh.at[dst_t], add=True)

        plsc.subcore_barrier()

        @pl.when(sid < NS - 1)
        def _():
            pltpu.sync_copy(acc_sh.at[pl.ds(sid * RPT, RPT)],
                            out_hbm.at[cid, pl.ds(sid * RPT, RPT)])

        @pl.when(sid == NS - 1)
        def _():
            pltpu.sync_copy(acc_sh.at[pl.ds((NS - 1) * RPT, RPT_LAST)],
                            out_hbm.at[cid, pl.ds((NS - 1) * RPT, RPT_LAST)])

    return _sc_aggregate


def _ci_dense_compute(p0, p1, xr, wlt, bl, wrt, br, g, be):
    agg = p0 + p1
    out = jnp.dot(agg, wlt, preferred_element_type=jnp.float32) + bl
    scale = jnp.dot(xr, wrt, preferred_element_type=jnp.float32) + br
    out = out * scale
    n = jnp.sqrt(jnp.sum(out * out, axis=-1, keepdims=True))
    out = out / jnp.maximum(n, 1e-12)
    out = jnp.maximum(out, 0.0)
    m = jnp.mean(out, axis=0, keepdims=True)
    v = jnp.mean((out - m) * (out - m), axis=0, keepdims=True)
    return (out - m) * jax.lax.rsqrt(v + 1e-5) * g + be


def _ci_dense_body(p0_ref, p1_ref, xr_ref, wlt_ref, bl_ref, wrt_ref, br_ref,
                   g_ref, be_ref, o_ref):
    o_ref[...] = _ci_dense_compute(
        p0_ref[...], p1_ref[...], xr_ref[...], wlt_ref[...], bl_ref[...],
        wrt_ref[...], br_ref[...], g_ref[...], be_ref[...])


def _dense2_head_body(p0_ref, p1_ref, xr_ref, wlt_ref, bl_ref, wrt_ref,
                      br_ref, g_ref, be_ref, w2t_ref, b2_ref, g3_ref, be3_ref,
                      w3t_ref, b3_ref, o_ref):
    x = _ci_dense_compute(
        p0_ref[...], p1_ref[...], xr_ref[...], wlt_ref[...], bl_ref[...],
        wrt_ref[...], br_ref[...], g_ref[...], be_ref[...])
    x = jnp.dot(x, w2t_ref[...], preferred_element_type=jnp.float32)
    x = jnp.maximum(x + b2_ref[...], 0.0)
    m = jnp.mean(x, axis=0, keepdims=True)
    v = jnp.mean((x - m) * (x - m), axis=0, keepdims=True)
    x = (x - m) * jax.lax.rsqrt(v + 1e-5) * g3_ref[...] + be3_ref[...]
    x = jnp.dot(x, w3t_ref[...], preferred_element_type=jnp.float32)
    x = jnp.maximum(x + b3_ref[...], 0.0)
    mx = jnp.max(x, axis=0, keepdims=True)
    ex = jnp.exp(x - mx)
    o_ref[...] = ex / jnp.sum(ex, axis=0, keepdims=True)


def _ci_dense(p, xr, Wl, bl, Wr, br, g, be):
    return pl.pallas_call(
        _ci_dense_body,
        out_shape=jax.ShapeDtypeStruct((N, Wl.shape[0]), jnp.float32),
    )(p[0], p[1], xr.reshape(1, -1), Wl.T, bl.reshape(1, -1), Wr.T,
      br.reshape(1, -1), g.reshape(1, -1), be.reshape(1, -1))


def kernel(node_feature, edge_index, global_x, Wl1, bl1, Wr1, br1, g1, be1,
           Wl2, bl2, Wr2, br2, g2, be2, W2, b2, g3, be3, W3, b3):
    zeros = jnp.zeros((RPT_LAST, D), jnp.float32)

    sc_aggregate = _make_sc_aggregate()
    eflat = edge_index.reshape(-1)
    p1 = sc_aggregate(node_feature, eflat, zeros)
    x1 = _ci_dense(p1, node_feature[1], Wl1, bl1, Wr1, br1, g1, be1)

    p2 = sc_aggregate(x1, eflat, zeros)

    return pl.pallas_call(
        _dense2_head_body,
        out_shape=jax.ShapeDtypeStruct((N, 1), jnp.float32),
    )(p2[0], p2[1], x1[1].reshape(1, -1), Wl2.T, bl2.reshape(1, -1), Wr2.T,
      br2.reshape(1, -1), g2.reshape(1, -1), be2.reshape(1, -1), W2.T,
      b2.reshape(1, -1), g3.reshape(1, -1), be3.reshape(1, -1), W3.T,
      b3.reshape(1, -1))


# speedup vs baseline: 1.0224x; 1.0224x over previous
import functools

import jax
import jax.numpy as jnp
from jax import lax
from jax.experimental import pallas as pl
from jax.experimental.pallas import tpu as pltpu
from jax.experimental.pallas import tpu_sc as plsc

N = 10000
E = 320000
D = 128

NC = 2
NS = 16
NW = NC * NS
EW = E // NW
CB = 64
FULL = EW // CB
TAIL = EW - FULL * CB
RPT = 624
RPT_LAST = N - RPT * (NS - 1)

NBUF = 4
STEPS = FULL // NBUF - 1


@functools.cache
def _make_sc_aggregate():
    mesh = plsc.VectorSubcoreMesh(
        core_axis_name="c", subcore_axis_name="s",
        num_cores=NC, num_subcores=NS,
    )

    @functools.partial(
        pl.kernel,
        out_type=jax.ShapeDtypeStruct((NC, N, D), jnp.float32),
        mesh=mesh,
        scratch_types=[
            pltpu.VMEM((FULL * CB,), jnp.int32),
            [pltpu.VMEM((CB,), jnp.int32) for _ in range(NBUF)],
            [pltpu.VMEM((CB, D), jnp.float32) for _ in range(NBUF)],
            [pltpu.SemaphoreType.DMA for _ in range(NBUF)],
            [pltpu.SemaphoreType.DMA for _ in range(NBUF)],
        ] + ([
            pltpu.VMEM((TAIL,), jnp.int32),
            pltpu.VMEM((TAIL,), jnp.int32),
            pltpu.VMEM((TAIL, D), jnp.float32),
            pltpu.SemaphoreType.DMA,
        ] if TAIL else []) + [
            pltpu.VMEM_SHARED((N, D), jnp.float32),
        ],
    )
    def _sc_aggregate(x_hbm, edge_hbm, zeros_hbm, out_hbm, src_v, dst_v,
                      rows, gsem, isem, *rest):
        if TAIL:
            src_t, dst_t, rows_t, sem, acc_sh = rest
        else:
            acc_sh, = rest
        cid = lax.axis_index("c")
        sid = lax.axis_index("s")
        wid = cid * NS + sid
        ebase = wid * EW

        pltpu.sync_copy(edge_hbm.at[pl.ds(ebase, FULL * CB)], src_v)
        if TAIL:
            pltpu.sync_copy(edge_hbm.at[pl.ds(ebase + FULL * CB, TAIL)],
                            src_t)
            pltpu.sync_copy(edge_hbm.at[pl.ds(E + ebase + FULL * CB, TAIL)],
                            dst_t)

        def issue(b, i):
            pltpu.async_copy(edge_hbm.at[pl.ds(E + ebase + i * CB, CB)],
                             dst_v[b], isem[b])
            pltpu.async_copy(x_hbm.at[src_v.at[pl.ds(i * CB, CB)]],
                             rows[b], gsem[b])

        def consume(b, i):
            pltpu.make_async_copy(edge_hbm.at[pl.ds(E + ebase + i * CB, CB)],
                                  dst_v[b], isem[b]).wait()
            pltpu.make_async_copy(x_hbm.at[src_v.at[pl.ds(i * CB, CB)]],
                                  rows[b], gsem[b]).wait()
            pltpu.sync_copy(rows[b], acc_sh.at[dst_v[b]], add=True)

        for b in range(NBUF):
            issue(b, b)
        if TAIL:
            pltpu.async_copy(x_hbm.at[src_t], rows_t, sem)

        @pl.when(sid < NS - 1)
        def _():
            pltpu.sync_copy(zeros_hbm.at[pl.ds(0, RPT)],
                            acc_sh.at[pl.ds(sid * RPT, RPT)])

        @pl.when(sid == NS - 1)
        def _():
            pltpu.sync_copy(zeros_hbm,
                            acc_sh.at[pl.ds((NS - 1) * RPT, RPT_LAST)])

        plsc.subcore_barrier()

        @pl.loop(0, STEPS)
        def _(j):
            i0 = j * NBUF
            for b in range(NBUF):
                consume(b, i0 + b)
                issue(b, i0 + NBUF + b)

        for k in range(STEPS * NBUF, FULL):
            b = k % NBUF
            consume(b, k)
            if k + NBUF < FULL:
                issue(b, k + NBUF)

        if TAIL:
            pltpu.make_async_copy(x_hbm.at[src_t], rows_t, sem).wait()
            pltpu.sync_copy(rows_t, acc_sh.at[dst_t], add=True)

        plsc.subcore_barrier()

        @pl.when(sid < NS - 1)
        def _():
            pltpu.sync_copy(acc_sh.at[pl.ds(sid * RPT, RPT)],
                            out_hbm.at[cid, pl.ds(sid * RPT, RPT)])

        @pl.when(sid == NS - 1)
        def _():
            pltpu.sync_copy(acc_sh.at[pl.ds((NS - 1) * RPT, RPT_LAST)],
                            out_hbm.at[cid, pl.ds((NS - 1) * RPT, RPT_LAST)])

    return _sc_aggregate


def _ci_dense_compute(p0, p1, xr, wlt, bl, wrt, br, g, be):
    agg = p0 + p1
    out = jnp.dot(agg, wlt, preferred_element_type=jnp.float32) + bl
    scale = jnp.dot(xr, wrt, preferred_element_type=jnp.float32) + br
    out = out * scale
    n = jnp.sqrt(jnp.sum(out * out, axis=-1, keepdims=True))
    out = out / jnp.maximum(n, 1e-12)
    out = jnp.maximum(out, 0.0)
    m = jnp.mean(out, axis=0, keepdims=True)
    v = jnp.mean((out - m) * (out - m), axis=0, keepdims=True)
    return (out - m) * jax.lax.rsqrt(v + 1e-5) * g + be


def _ci_dense_body(p0_ref, p1_ref, xr_ref, wlt_ref, bl_ref, wrt_ref, br_ref,
                   g_ref, be_ref, o_ref):
    o_ref[...] = _ci_dense_compute(
        p0_ref[...], p1_ref[...], xr_ref[...], wlt_ref[...], bl_ref[...],
        wrt_ref[...], br_ref[...], g_ref[...], be_ref[...])


def _dense2_head_body(p0_ref, p1_ref, xr_ref, wlt_ref, bl_ref, wrt_ref,
                      br_ref, g_ref, be_ref, w2t_ref, b2_ref, g3_ref, be3_ref,
                      w3t_ref, b3_ref, o_ref):
    x = _ci_dense_compute(
        p0_ref[...], p1_ref[...], xr_ref[...], wlt_ref[...], bl_ref[...],
        wrt_ref[...], br_ref[...], g_ref[...], be_ref[...])
    x = jnp.dot(x, w2t_ref[...], preferred_element_type=jnp.float32)
    x = jnp.maximum(x + b2_ref[...], 0.0)
    m = jnp.mean(x, axis=0, keepdims=True)
    v = jnp.mean((x - m) * (x - m), axis=0, keepdims=True)
    x = (x - m) * jax.lax.rsqrt(v + 1e-5) * g3_ref[...] + be3_ref[...]
    x = jnp.dot(x, w3t_ref[...], preferred_element_type=jnp.float32)
    x = jnp.maximum(x + b3_ref[...], 0.0)
    mx = jnp.max(x, axis=0, keepdims=True)
    ex = jnp.exp(x - mx)
    o_ref[...] = ex / jnp.sum(ex, axis=0, keepdims=True)


def _ci_dense(p, xr, Wl, bl, Wr, br, g, be):
    return pl.pallas_call(
        _ci_dense_body,
        out_shape=jax.ShapeDtypeStruct((N, Wl.shape[0]), jnp.float32),
    )(p[0], p[1], xr.reshape(1, -1), Wl.T, bl.reshape(1, -1), Wr.T,
      br.reshape(1, -1), g.reshape(1, -1), be.reshape(1, -1))


def kernel(node_feature, edge_index, global_x, Wl1, bl1, Wr1, br1, g1, be1,
           Wl2, bl2, Wr2, br2, g2, be2, W2, b2, g3, be3, W3, b3):
    zeros = jnp.zeros((RPT_LAST, D), jnp.float32)

    sc_aggregate = _make_sc_aggregate()
    eflat = edge_index.reshape(-1)
    p1 = sc_aggregate(node_feature, eflat, zeros)
    x1 = _ci_dense(p1, node_feature[1], Wl1, bl1, Wr1, br1, g1, be1)

    p2 = sc_aggregate(x1, eflat, zeros)

    return pl.pallas_call(
        _dense2_head_body,
        out_shape=jax.ShapeDtypeStruct((N, 1), jnp.float32),
    )(p2[0], p2[1], x1[1].reshape(1, -1), Wl2.T, bl2.reshape(1, -1), Wr2.T,
      br2.reshape(1, -1), g2.reshape(1, -1), be2.reshape(1, -1), W2.T,
      b2.reshape(1, -1), g3.reshape(1, -1), be3.reshape(1, -1), W3.T,
      b3.reshape(1, -1))
